# baseline (speedup 1.0000x reference)
import functools

import jax
import jax.numpy as jnp
from jax.experimental import pallas as pl
from jax.experimental.pallas import tpu as pltpu

_B = 4096
_BB = 512
_GRID = _B // _BB
_NLEV = 4
_NEMB = 1024


def _mlp_fwd(h, wbs):
    n = len(wbs)
    for i, (w, b) in enumerate(wbs):
        h = jnp.dot(h, w, preferred_element_type=jnp.float32) + b
        if i < n - 1:
            h = jax.nn.relu(h)
    return h


def _rvq_block(e, cb_ref, idx_out, loss_out):
    residual = e
    xq = jnp.zeros_like(e)
    err2_acc = jnp.zeros_like(e)
    col_iota = jax.lax.broadcasted_iota(jnp.int32, (_BB, _NEMB), 1)
    idxs = []
    for level in range(_NLEV):
        cb = cb_ref[level]
        d = (jnp.sum(residual ** 2, axis=1, keepdims=True)
             - 2.0 * jax.lax.dot_general(residual, cb, (((1,), (1,)), ((), ())),
                                         preferred_element_type=jnp.float32)
             + jnp.sum(cb ** 2, axis=1)[None, :])
        dmin = jnp.min(d, axis=1, keepdims=True)
        idx = jnp.min(jnp.where(d == dmin, col_iota, _NEMB), axis=1, keepdims=True)
        onehot = (col_iota == idx).astype(jnp.float32)
        q = jax.lax.dot_general(onehot, cb, (((1,), (0,)), ((), ())),
                                precision=jax.lax.Precision.HIGHEST,
                                preferred_element_type=jnp.float32)
        err = residual - q
        err2_acc = err2_acc + err * err
        idxs.append(idx)
        xq = xq + q
        residual = err
    idx_out[...] = jnp.concatenate(idxs, axis=1)
    loss_out[...] = jnp.sum(err2_acc, axis=0, keepdims=True)[None]
    return xq


def _fused_kernel(x_ref, y_ref,
                  te0w, te0b, te1w, te1b, te2w, te2b,
                  ke0w, ke0b, ke1w, ke1b, ke2w, ke2b,
                  td0w, td0b, td1w, td1b, td2w, td2b,
                  kd0w, kd0b, kd1w, kd1b, kd2w, kd2b,
                  tcb_ref, kcb_ref,
                  tout_ref, kout_ref, xq_ref, yq_ref,
                  idx_ref, idx2_ref, loss_ref, loss2_ref):
    te = [(te0w[...], te0b[...]), (te1w[...], te1b[...]), (te2w[...], te2b[...])]
    ke = [(ke0w[...], ke0b[...]), (ke1w[...], ke1b[...]), (ke2w[...], ke2b[...])]
    td = [(td0w[...], td0b[...]), (td1w[...], td1b[...]), (td2w[...], td2b[...])]
    kd = [(kd0w[...], kd0b[...]), (kd1w[...], kd1b[...]), (kd2w[...], kd2b[...])]

    xe = _mlp_fwd(x_ref[...], te)
    xq = _rvq_block(xe, tcb_ref, idx_ref, loss_ref)
    xq_ref[...] = xq
    tout_ref[...] = _mlp_fwd(xq, td)

    ye = _mlp_fwd(y_ref[...], ke)
    yq = _rvq_block(ye, kcb_ref, idx2_ref, loss2_ref)
    yq_ref[...] = yq
    kout_ref[...] = _mlp_fwd(yq, kd)


@functools.partial(jax.jit, static_argnums=())
def kernel(x, y, labels, labels_2, params):
    del labels, labels_2
    te = params['text_enc']
    ke = params['kg_enc']
    td = params['text_dec']
    kd = params['kg_dec']
    tcb = params['text_cb']
    kcb = params['kg_cb']

    def wb(pairs):
        out = []
        for w, b in pairs:
            out.append(w)
            out.append(b.reshape(1, -1))
        return out

    operands = ([x, y] + wb(te) + wb(ke) + wb(td) + wb(kd) + [tcb, kcb])

    def data_spec(cols):
        return pl.BlockSpec((_BB, cols), lambda i: (i, 0))

    def full_spec(a):
        nd = a.ndim
        return pl.BlockSpec(a.shape, lambda i, _nd=nd: (0,) * _nd)

    in_specs = ([data_spec(x.shape[1]), data_spec(y.shape[1])]
                + [full_spec(a) for a in operands[2:]])

    out_shapes = (
        jax.ShapeDtypeStruct((_B, x.shape[1]), jnp.float32),
        jax.ShapeDtypeStruct((_B, y.shape[1]), jnp.float32),
        jax.ShapeDtypeStruct((_B, tcb.shape[2]), jnp.float32),
        jax.ShapeDtypeStruct((_B, kcb.shape[2]), jnp.float32),
        jax.ShapeDtypeStruct((_B, _NLEV), jnp.int32),
        jax.ShapeDtypeStruct((_B, _NLEV), jnp.int32),
        jax.ShapeDtypeStruct((_GRID, 1, 64), jnp.float32),
        jax.ShapeDtypeStruct((_GRID, 1, 64), jnp.float32),
    )
    out_specs = (
        data_spec(x.shape[1]),
        data_spec(y.shape[1]),
        data_spec(tcb.shape[2]),
        data_spec(kcb.shape[2]),
        pl.BlockSpec((_BB, _NLEV), lambda i: (i, 0)),
        pl.BlockSpec((_BB, _NLEV), lambda i: (i, 0)),
        pl.BlockSpec((1, 1, 64), lambda i: (i, 0, 0)),
        pl.BlockSpec((1, 1, 64), lambda i: (i, 0, 0)),
    )

    tout, kout, xq, yq, idx, idx2, loss_sums, loss2_sums = pl.pallas_call(
        _fused_kernel,
        grid=(_GRID,),
        in_specs=in_specs,
        out_specs=out_specs,
        out_shape=out_shapes,
        compiler_params=pltpu.CompilerParams(
            dimension_semantics=("parallel",)),
    )(*operands)

    denom = jnp.float32(_B * tcb.shape[2] * _NLEV)
    rq_loss = 1.25 * jnp.sum(loss_sums) / denom
    rq_loss_2 = 1.25 * jnp.sum(loss2_sums) / denom
    return (tout, kout, rq_loss, rq_loss_2, idx, idx2, xq, yq)


# speedup vs baseline: 1.4144x; 1.4144x over previous
import functools

import jax
import jax.numpy as jnp
from jax.experimental import pallas as pl
from jax.experimental.pallas import tpu as pltpu

_B = 4096
_BB = 512
_GRID = _B // _BB
_NLEV = 4
_NEMB = 1024


def _mlp_fwd(h, wbs):
    n = len(wbs)
    for i, (w, b) in enumerate(wbs):
        h = jnp.dot(h, w, preferred_element_type=jnp.float32) + b
        if i < n - 1:
            h = jax.nn.relu(h)
    return h


def _rvq_block(e, cb_ref, cbs_ref, idx_out, loss_out):
    residual = e
    xq = jnp.zeros_like(e)
    err2_acc = jnp.zeros_like(e)
    col_iota = jax.lax.broadcasted_iota(jnp.int32, (_BB, _NEMB), 1)
    idxs = []
    for level in range(_NLEV):
        cb = cb_ref[level]
        d = (jnp.sum(residual ** 2, axis=1, keepdims=True)
             - 2.0 * jax.lax.dot_general(residual, cb, (((1,), (1,)), ((), ())),
                                         preferred_element_type=jnp.float32)
             + jnp.sum(cb ** 2, axis=1)[None, :])
        dmin = jnp.min(d, axis=1, keepdims=True)
        idx = jnp.min(jnp.where(d == dmin, col_iota, _NEMB), axis=1, keepdims=True)
        onehot = (col_iota == idx).astype(jnp.bfloat16)
        parts = [
            jax.lax.dot_general(onehot, cbs_ref[s, level],
                                (((1,), (0,)), ((), ())),
                                preferred_element_type=jnp.float32)
            for s in range(3)
        ]
        q = (parts[0] + parts[1]) + parts[2]
        err = residual - q
        err2_acc = err2_acc + err * err
        idxs.append(idx)
        xq = xq + q
        residual = err
    idx_out[...] = jnp.concatenate(idxs, axis=1)
    loss_out[...] = jnp.sum(err2_acc, axis=0, keepdims=True)[None]
    return xq


def _fused_kernel(x_ref, y_ref,
                  te0w, te0b, te1w, te1b, te2w, te2b,
                  ke0w, ke0b, ke1w, ke1b, ke2w, ke2b,
                  td0w, td0b, td1w, td1b, td2w, td2b,
                  kd0w, kd0b, kd1w, kd1b, kd2w, kd2b,
                  tcb_ref, kcb_ref, tcbs_ref, kcbs_ref,
                  tout_ref, kout_ref, xq_ref, yq_ref,
                  idx_ref, idx2_ref, loss_ref, loss2_ref):
    te = [(te0w[...], te0b[...]), (te1w[...], te1b[...]), (te2w[...], te2b[...])]
    ke = [(ke0w[...], ke0b[...]), (ke1w[...], ke1b[...]), (ke2w[...], ke2b[...])]
    td = [(td0w[...], td0b[...]), (td1w[...], td1b[...]), (td2w[...], td2b[...])]
    kd = [(kd0w[...], kd0b[...]), (kd1w[...], kd1b[...]), (kd2w[...], kd2b[...])]

    xe = _mlp_fwd(x_ref[...], te)
    xq = _rvq_block(xe, tcb_ref, tcbs_ref, idx_ref, loss_ref)
    xq_ref[...] = xq
    tout_ref[...] = _mlp_fwd(xq, td)

    ye = _mlp_fwd(y_ref[...], ke)
    yq = _rvq_block(ye, kcb_ref, kcbs_ref, idx2_ref, loss2_ref)
    yq_ref[...] = yq
    kout_ref[...] = _mlp_fwd(yq, kd)


@functools.partial(jax.jit, static_argnums=())
def kernel(x, y, labels, labels_2, params):
    del labels, labels_2
    te = params['text_enc']
    ke = params['kg_enc']
    td = params['text_dec']
    kd = params['kg_dec']
    tcb = params['text_cb']
    kcb = params['kg_cb']

    def wb(pairs):
        out = []
        for w, b in pairs:
            out.append(w)
            out.append(b.reshape(1, -1))
        return out

    def split3(cb):
        p1 = cb.astype(jnp.bfloat16)
        r1 = cb - p1.astype(jnp.float32)
        p2 = r1.astype(jnp.bfloat16)
        p3 = (r1 - p2.astype(jnp.float32)).astype(jnp.bfloat16)
        return jnp.stack([p1, p2, p3])

    operands = ([x, y] + wb(te) + wb(ke) + wb(td) + wb(kd)
                + [tcb, kcb, split3(tcb), split3(kcb)])

    def data_spec(cols):
        return pl.BlockSpec((_BB, cols), lambda i: (i, 0))

    def full_spec(a):
        nd = a.ndim
        return pl.BlockSpec(a.shape, lambda i, _nd=nd: (0,) * _nd)

    in_specs = ([data_spec(x.shape[1]), data_spec(y.shape[1])]
                + [full_spec(a) for a in operands[2:]])

    out_shapes = (
        jax.ShapeDtypeStruct((_B, x.shape[1]), jnp.float32),
        jax.ShapeDtypeStruct((_B, y.shape[1]), jnp.float32),
        jax.ShapeDtypeStruct((_B, tcb.shape[2]), jnp.float32),
        jax.ShapeDtypeStruct((_B, kcb.shape[2]), jnp.float32),
        jax.ShapeDtypeStruct((_B, _NLEV), jnp.int32),
        jax.ShapeDtypeStruct((_B, _NLEV), jnp.int32),
        jax.ShapeDtypeStruct((_GRID, 1, 64), jnp.float32),
        jax.ShapeDtypeStruct((_GRID, 1, 64), jnp.float32),
    )
    out_specs = (
        data_spec(x.shape[1]),
        data_spec(y.shape[1]),
        data_spec(tcb.shape[2]),
        data_spec(kcb.shape[2]),
        pl.BlockSpec((_BB, _NLEV), lambda i: (i, 0)),
        pl.BlockSpec((_BB, _NLEV), lambda i: (i, 0)),
        pl.BlockSpec((1, 1, 64), lambda i: (i, 0, 0)),
        pl.BlockSpec((1, 1, 64), lambda i: (i, 0, 0)),
    )

    tout, kout, xq, yq, idx, idx2, loss_sums, loss2_sums = pl.pallas_call(
        _fused_kernel,
        grid=(_GRID,),
        in_specs=in_specs,
        out_specs=out_specs,
        out_shape=out_shapes,
        compiler_params=pltpu.CompilerParams(
            dimension_semantics=("parallel",)),
    )(*operands)

    denom = jnp.float32(_B * tcb.shape[2] * _NLEV)
    rq_loss = 1.25 * jnp.sum(loss_sums) / denom
    rq_loss_2 = 1.25 * jnp.sum(loss2_sums) / denom
    return (tout, kout, rq_loss, rq_loss_2, idx, idx2, xq, yq)
